# Initial kernel scaffold; baseline (speedup 1.0000x reference)
import jax
import jax.numpy as jnp
from jax.experimental import pallas as pl


def kernel(logits, temperatures, top_ks, top_ps, min_ps):
    raise NotImplementedError("write your pallas kernel here")


# speedup vs baseline: 10.4541x; 10.4541x over previous
import jax
import jax.numpy as jnp
import numpy as np
from jax import lax
from jax.experimental import pallas as pl
from jax.experimental.pallas import tpu as pltpu

B = 128
V = 100000
NCH = 782
VP = NCH * 128
ROUNDS = 12
NEXT = ROUNDS * 128
NSORT = 2048
NCAND = 256
NEG_INF = float("-inf")


def _extract_body(x_ref, vals_ref, idx_ref, maxl_ref):
    lane = lax.broadcasted_iota(jnp.int32, (8, 128), 1)
    pv = jnp.full((8, 128), jnp.inf, jnp.float32)
    pc = jnp.full((8, 128), -1, jnp.int32)
    for r in range(ROUNDS):
        def scan(c, carry):
            lm, lc = carry
            x = x_ref[:, pl.ds(c, 1), :].reshape(8, 128)
            elig = (x < pv) | ((x == pv) & (c > pc))
            nm = jnp.where(elig, x, NEG_INF)
            better = nm > lm
            lm = jnp.where(better, nm, lm)
            lc = jnp.where(better, jnp.full((8, 128), c, jnp.int32), lc)
            return lm, lc
        lm, lc = lax.fori_loop(
            0, NCH, scan,
            (jnp.full((8, 128), NEG_INF, jnp.float32),
             jnp.full((8, 128), NCH, jnp.int32)))
        vals_ref[:, pl.ds(r * 128, 128)] = lm
        idx_ref[:, pl.ds(r * 128, 128)] = lc * 128 + lane
        if r == 0:
            maxl_ref[...] = jnp.broadcast_to(
                jnp.max(lm, axis=1, keepdims=True), (8, 128))
        pv, pc = lm, lc


_extract = pl.pallas_call(
    _extract_body,
    grid=(16,),
    in_specs=[pl.BlockSpec((8, NCH, 128), lambda b: (b, 0, 0))],
    out_specs=[pl.BlockSpec((8, NEXT), lambda b: (b, 0)),
               pl.BlockSpec((8, NEXT), lambda b: (b, 0)),
               pl.BlockSpec((8, 128), lambda b: (b, 0))],
    out_shape=[jax.ShapeDtypeStruct((B, NEXT), jnp.float32),
               jax.ShapeDtypeStruct((B, NEXT), jnp.int32),
               jax.ShapeDtypeStruct((B, 128), jnp.float32)],
)


def _sort_body(v_ref, i_ref, vo_ref, io_ref):
    x = v_ref[...]
    ix = i_ref[...]
    sub = lax.broadcasted_iota(jnp.int32, (NSORT, 128), 0)
    k = 2
    while k <= NSORT:
        j = k // 2
        while j >= 1:
            pv = pltpu.roll(x, NSORT - j, 0)
            nv = pltpu.roll(x, j, 0)
            pi = pltpu.roll(ix, NSORT - j, 0)
            ni = pltpu.roll(ix, j, 0)
            low = (sub & j) == 0
            part_v = jnp.where(low, pv, nv)
            part_i = jnp.where(low, pi, ni)
            desc = (sub & k) == 0
            take_hi = low == desc
            t = jnp.where(take_hi, jnp.maximum(x, part_v),
                          jnp.minimum(x, part_v))
            ix = jnp.where(t == x, ix, part_i)
            x = t
            j //= 2
        k *= 2
    vo_ref[...] = x[:NCAND]
    io_ref[...] = ix[:NCAND]


_sort = pl.pallas_call(
    _sort_body,
    grid=(1,),
    in_specs=[pl.BlockSpec((NSORT, 128), lambda i: (0, 0)),
              pl.BlockSpec((NSORT, 128), lambda i: (0, 0))],
    out_specs=[pl.BlockSpec((NCAND, 128), lambda i: (0, 0)),
               pl.BlockSpec((NCAND, 128), lambda i: (0, 0))],
    out_shape=[jax.ShapeDtypeStruct((NCAND, 128), jnp.float32),
               jax.ShapeDtypeStruct((NCAND, 128), jnp.int32)],
)


def _threefry_gumbel(flat):
    k1 = jnp.uint32(0)
    k2 = jnp.uint32(42)
    x0 = jnp.zeros_like(flat, dtype=jnp.uint32)
    x1 = flat.astype(jnp.uint32)
    rot = [[13, 15, 26, 6], [17, 29, 16, 24]]
    ks = [k1, k2, k1 ^ k2 ^ jnp.uint32(0x1BD11BDA)]
    x0 = x0 + ks[0]
    x1 = x1 + ks[1]
    for i in range(5):
        for r in rot[i % 2]:
            x0 = x0 + x1
            x1 = (x1 << jnp.uint32(r)) | (x1 >> jnp.uint32(32 - r))
            x1 = x0 ^ x1
        x0 = x0 + ks[(i + 1) % 3]
        x1 = x1 + ks[(i + 2) % 3] + jnp.uint32(i + 1)
    bits = x0 ^ x1
    fb = (bits >> jnp.uint32(9)) | jnp.uint32(0x3F800000)
    u = lax.bitcast_convert_type(fb, jnp.float32) - jnp.float32(1.0)
    tiny = jnp.float32(np.finfo(np.float32).tiny)
    u = jnp.maximum(tiny, u * (jnp.float32(1.0) - tiny) + tiny)
    return -jnp.log(-jnp.log(u))


def _cand_body(vals_ref, idx_ref, maxl_ref, cnt_ref, temp_ref, k_ref,
               topp_ref, minp_ref, out_ref, q_ref):
    slot = lax.broadcasted_iota(jnp.int32, (NCAND, B), 0)
    valid = slot < cnt_ref[...]
    vals = jnp.where(valid, vals_ref[...], NEG_INF)
    idx = idx_ref[...]
    t = temp_ref[...]
    e = jnp.exp(vals / t - maxl_ref[...] / t)

    u = lax.bitcast_convert_type(e, jnp.int32)
    k = k_ref[...]

    def bitloop(b, res):
        cand = res | (jnp.int32(1) << (30 - b))
        c = jnp.sum((u >= cand).astype(jnp.int32), axis=0, keepdims=True)
        return jnp.where(c >= k, cand, res)
    uk = lax.fori_loop(0, 31, bitloop, jnp.zeros((1, B), jnp.int32))
    keptA = u >= uk
    eA = jnp.where(keptA, e, 0.0)
    q = eA / jnp.sum(eA, axis=0, keepdims=True)
    q_ref[...] = q

    def massloop(l, acc):
        ql = q_ref[pl.ds(l, 1), :]
        il = idx_ref[pl.ds(l, 1), :]
        before = (ql > q) | ((ql == q) & (il < idx))
        return acc + jnp.where(before, ql, 0.0)
    mass = lax.fori_loop(0, NCAND, massloop,
                         jnp.zeros((NCAND, B), jnp.float32))
    keep2 = mass < topp_ref[...]
    m2 = jnp.where(keep2, q, 0.0)
    r = m2 / jnp.sum(m2, axis=0, keepdims=True)

    keep3 = r >= minp_ref[...] * jnp.max(r, axis=0, keepdims=True)
    m3 = jnp.where(keep3, r, 0.0)
    r3 = m3 / jnp.sum(m3, axis=0, keepdims=True)
    lg = jnp.log(r3)

    row = lax.broadcasted_iota(jnp.int32, (NCAND, B), 1)
    g = _threefry_gumbel(row * V + idx)
    score = lg + g
    wm = jnp.max(score, axis=0, keepdims=True)
    win = jnp.min(jnp.where(score == wm, idx, jnp.int32(V + 1)),
                  axis=0, keepdims=True)
    out_ref[...] = jnp.broadcast_to(win, (8, B))


_cand_stage = pl.pallas_call(
    _cand_body,
    grid=(1,),
    in_specs=[
        pl.BlockSpec((NCAND, B), lambda i: (0, 0)),
        pl.BlockSpec((NCAND, B), lambda i: (0, 0)),
        pl.BlockSpec((1, B), lambda i: (0, 0)),
        pl.BlockSpec((1, B), lambda i: (0, 0)),
        pl.BlockSpec((1, B), lambda i: (0, 0)),
        pl.BlockSpec((1, B), lambda i: (0, 0)),
        pl.BlockSpec((1, B), lambda i: (0, 0)),
        pl.BlockSpec((1, B), lambda i: (0, 0)),
    ],
    out_specs=pl.BlockSpec((8, B), lambda i: (0, 0)),
    out_shape=jax.ShapeDtypeStruct((8, B), jnp.int32),
    scratch_shapes=[
        pltpu.VMEM((NCAND, B), jnp.float32),
    ],
)


def kernel(logits, temperatures, top_ks, top_ps, min_ps):
    logits = logits.astype(jnp.float32)
    x3 = jnp.pad(logits, ((0, 0), (0, VP - V)),
                 constant_values=NEG_INF).reshape(B, NCH, 128)
    vals, idxs, maxl = _extract(x3)
    vpad = jnp.pad(vals, ((0, 0), (0, NSORT - NEXT)),
                   constant_values=NEG_INF)
    ipad = jnp.pad(idxs, ((0, 0), (0, NSORT - NEXT)))
    vs, is_ = _sort(vpad.T, ipad.T)
    out = _cand_stage(
        vs, is_,
        maxl[:, 0].reshape(1, B),
        jnp.full((1, B), NCAND, jnp.int32),
        temperatures.astype(jnp.float32).reshape(1, B),
        top_ks.astype(jnp.int32).reshape(1, B),
        top_ps.astype(jnp.float32).reshape(1, B),
        min_ps.astype(jnp.float32).reshape(1, B),
    )
    return out[0]
